# Initial kernel scaffold; baseline (speedup 1.0000x reference)
import jax
import jax.numpy as jnp
from jax.experimental import pallas as pl


def kernel(f_nodes, f_edges, W_i, W_h, W_o, b_o, n2e, e2n, e2reversee, mol_ids):
    raise NotImplementedError("write your pallas kernel here")


# speedup vs baseline: 1.4546x; 1.4546x over previous
import functools

import jax
import jax.numpy as jnp
from jax import lax
from jax.experimental import pallas as pl
from jax.experimental.pallas import tpu as pltpu
from jax.experimental.pallas import tpu_sc as plsc

NMOL = 256
CH = 128
NW = 32


def _mm_relu_body(x_ref, w_ref, inp_ref, msg_ref):
    acc = jnp.dot(x_ref[...], w_ref[...], preferred_element_type=jnp.float32)
    inp_ref[...] = acc
    msg_ref[...] = jnp.maximum(acc, 0.0)


def _edge_init(f_edges, W_i, rows_per_block):
    e, ef = f_edges.shape
    h = W_i.shape[1]
    grid = e // rows_per_block
    return pl.pallas_call(
        _mm_relu_body,
        grid=(grid,),
        in_specs=[
            pl.BlockSpec((rows_per_block, ef), lambda i: (i, 0)),
            pl.BlockSpec((ef, h), lambda i: (0, 0)),
        ],
        out_specs=[
            pl.BlockSpec((rows_per_block, h), lambda i: (i, 0)),
            pl.BlockSpec((rows_per_block, h), lambda i: (i, 0)),
        ],
        out_shape=[jax.ShapeDtypeStruct((e, h), jnp.float32)] * 2,
    )(f_edges, W_i)


def _mm_body(x_ref, w_ref, o_ref):
    o_ref[...] = jnp.dot(x_ref[...], w_ref[...], preferred_element_type=jnp.float32)


def _matmul(x, w, rows_per_block):
    m, k = x.shape
    h = w.shape[1]
    grid = m // rows_per_block
    return pl.pallas_call(
        _mm_body,
        grid=(grid,),
        in_specs=[
            pl.BlockSpec((rows_per_block, k), lambda i: (i, 0)),
            pl.BlockSpec((k, h), lambda i: (0, 0)),
        ],
        out_specs=pl.BlockSpec((rows_per_block, h), lambda i: (i, 0)),
        out_shape=jax.ShapeDtypeStruct((m, h), jnp.float32),
    )(x, w)


def _readout_body(fn_ref, nm_ref, wo_ref, bo_ref, mol_ref, out_ref,
                  sum_acc, cnt_acc):
    i = pl.program_id(0)
    n_steps = pl.num_programs(0)
    a = jnp.concatenate([fn_ref[...], nm_ref[...]], axis=1)
    h = jnp.dot(a, wo_ref[...], preferred_element_type=jnp.float32)
    h = jnp.maximum(h + bo_ref[...], 0.0)
    mol = mol_ref[0, 0, :]
    rows = mol.shape[0]
    iota = lax.broadcasted_iota(jnp.int32, (NMOL, rows), 0)
    onehot = (mol[None, :] == iota).astype(jnp.float32)
    psum = jnp.dot(onehot, h, preferred_element_type=jnp.float32)
    pcnt = jnp.sum(onehot, axis=1, keepdims=True)

    @pl.when(i == 0)
    def _():
        sum_acc[...] = jnp.zeros_like(sum_acc)
        cnt_acc[...] = jnp.zeros_like(cnt_acc)

    sum_acc[...] += psum
    cnt_acc[...] += jnp.broadcast_to(pcnt, cnt_acc.shape)

    @pl.when(i == n_steps - 1)
    def _():
        out_ref[...] = sum_acc[...] / jnp.maximum(cnt_acc[...], 1.0)


def _readout(f_nodes, nm, W_o, b_o, mol_ids, rows_per_block):
    n, nf = f_nodes.shape
    h = W_o.shape[1]
    grid = n // rows_per_block
    mol3 = mol_ids.reshape(grid, 1, rows_per_block)
    return pl.pallas_call(
        _readout_body,
        grid=(grid,),
        in_specs=[
            pl.BlockSpec((rows_per_block, nf), lambda i: (i, 0)),
            pl.BlockSpec((rows_per_block, h), lambda i: (i, 0)),
            pl.BlockSpec(W_o.shape, lambda i: (0, 0)),
            pl.BlockSpec((1, h), lambda i: (0, 0)),
            pl.BlockSpec((1, 1, rows_per_block), lambda i: (i, 0, 0)),
        ],
        out_specs=pl.BlockSpec((NMOL, h), lambda i: (0, 0)),
        out_shape=jax.ShapeDtypeStruct((NMOL, h), jnp.float32),
        scratch_shapes=[
            pltpu.VMEM((NMOL, h), jnp.float32),
            pltpu.VMEM((NMOL, h), jnp.float32),
        ],
    )(f_nodes, nm, W_o, b_o.reshape(1, h), mol3)


def _seg_sum(msg, n2e_flat, n, deg):
    e, h = msg.shape
    ng = h // 16
    nodes_per_chunk = CH // deg
    n_chunks = (n * deg) // CH
    iters = (n_chunks + NW - 1) // NW

    mesh = plsc.VectorSubcoreMesh(core_axis_name="c", subcore_axis_name="s")

    @functools.partial(
        pl.kernel, mesh=mesh,
        out_type=jax.ShapeDtypeStruct((n, h), jnp.float32),
        scratch_types=[
            pltpu.VMEM((CH,), jnp.int32),
            pltpu.VMEM((CH, h), jnp.float32),
            pltpu.VMEM((nodes_per_chunk, h), jnp.float32),
            pltpu.SemaphoreType.DMA,
        ],
    )
    def seg_kernel(msg_hbm, idx_hbm, out_hbm, idx_v, rows_v, acc_v, sem):
        wid = lax.axis_index("s") * 2 + lax.axis_index("c")

        def chunk_body(i, _):
            c = i * NW + wid

            @pl.when(c < n_chunks)
            def _():
                pltpu.sync_copy(idx_hbm.at[pl.ds(c * CH, CH)], idx_v)
                pltpu.async_copy(msg_hbm.at[idx_v], rows_v, sem).wait()

                def row_body(r, accs):
                    out = []
                    for j in range(nodes_per_chunk):
                        for g in range(ng):
                            v = rows_v[j * deg + r, pl.ds(g * 16, 16)]
                            out.append(accs[j * ng + g] + v)
                    return tuple(out)

                accs = tuple(
                    jnp.zeros((16,), jnp.float32)
                    for _ in range(nodes_per_chunk * ng))
                accs = lax.fori_loop(0, deg, row_body, accs)
                for j in range(nodes_per_chunk):
                    for g in range(ng):
                        acc_v[j, pl.ds(g * 16, 16)] = accs[j * ng + g]
                pltpu.sync_copy(
                    acc_v, out_hbm.at[pl.ds(c * nodes_per_chunk,
                                            nodes_per_chunk)])
            return 0

        lax.fori_loop(0, iters, chunk_body, 0)

    return seg_kernel(msg, n2e_flat)


def _edge_update(inp, nm2, m2, e2n, e2rev):
    e, h = inp.shape
    ng = h // 16
    n_chunks = e // CH
    iters = (n_chunks + NW - 1) // NW

    mesh = plsc.VectorSubcoreMesh(core_axis_name="c", subcore_axis_name="s")

    @functools.partial(
        pl.kernel, mesh=mesh,
        out_type=jax.ShapeDtypeStruct((e, h), jnp.float32),
        scratch_types=[
            pltpu.VMEM((CH,), jnp.int32),
            pltpu.VMEM((CH,), jnp.int32),
            pltpu.VMEM((CH, h), jnp.float32),
            pltpu.VMEM((CH, h), jnp.float32),
            pltpu.VMEM((CH, h), jnp.float32),
            pltpu.VMEM((CH, h), jnp.float32),
            pltpu.SemaphoreType.DMA,
            pltpu.SemaphoreType.DMA,
            pltpu.SemaphoreType.DMA,
        ],
    )
    def upd_kernel(inp_hbm, nm2_hbm, m2_hbm, e2n_hbm, rev_hbm, out_hbm,
                   idx1_v, idx2_v, a_v, b_v, c_v, o_v, sem1, sem2, sem3):
        wid = lax.axis_index("s") * 2 + lax.axis_index("c")

        def chunk_body(i, _):
            c = i * NW + wid

            @pl.when(c < n_chunks)
            def _():
                base = c * CH
                pltpu.sync_copy(e2n_hbm.at[pl.ds(base, CH)], idx1_v)
                pltpu.sync_copy(rev_hbm.at[pl.ds(base, CH)], idx2_v)
                cp1 = pltpu.async_copy(nm2_hbm.at[idx1_v], a_v, sem1)
                cp2 = pltpu.async_copy(m2_hbm.at[idx2_v], b_v, sem2)
                cp3 = pltpu.async_copy(inp_hbm.at[pl.ds(base, CH)], c_v, sem3)
                cp1.wait()
                cp2.wait()
                cp3.wait()

                def row_body(r, carry):
                    for g in range(ng):
                        sl = pl.ds(g * 16, 16)
                        v = c_v[r, sl] + a_v[r, sl] - b_v[r, sl]
                        o_v[r, sl] = jnp.maximum(v, 0.0)
                    return carry

                lax.fori_loop(0, CH, row_body, 0)
                pltpu.sync_copy(o_v, out_hbm.at[pl.ds(base, CH)])
            return 0

        lax.fori_loop(0, iters, chunk_body, 0)

    return upd_kernel(inp, nm2, m2, e2n, e2rev)


def kernel(f_nodes, f_edges, W_i, W_h, W_o, b_o, n2e, e2n, e2reversee,
           mol_ids):
    n, deg = n2e.shape
    e = f_edges.shape[0]
    n2e_flat = n2e.reshape(-1)

    inp, msg = _edge_init(f_edges, W_i, rows_per_block=2000)
    for _ in range(2):
        nm = _seg_sum(msg, n2e_flat, n, deg)
        m2 = _matmul(msg, W_h, rows_per_block=2000)
        nm2 = _matmul(nm, W_h, rows_per_block=1000)
        msg = _edge_update(inp, nm2, m2, e2n, e2reversee)
    nm = _seg_sum(msg, n2e_flat, n, deg)
    return _readout(f_nodes, nm, W_o, b_o, mol_ids, rows_per_block=1000)
